# Initial kernel scaffold; baseline (speedup 1.0000x reference)
import jax
import jax.numpy as jnp
from jax.experimental import pallas as pl


def kernel(g_in, W_ih, b_ih, W_hh, b_hh, Wg, bg, Wm, W_mu, b_mu, W_lv, b_lv):
    raise NotImplementedError("write your pallas kernel here")


# speedup vs baseline: 7.0965x; 7.0965x over previous
import jax
import jax.numpy as jnp
from jax.experimental import pallas as pl
from jax.experimental.pallas import tpu as pltpu

N = 200
H = 200
Z = 56

_DN_T = (((1,), (1,)), ((), ()))


def _encode_kernel(adjT_ref, types_ref, Wih_ref, bih_ref, Whh_ref, bhh_ref,
                   Wg_ref, bg_ref, Wm_ref, Wmu_ref, bmu_ref, Wlv_ref, blv_ref,
                   mu_ref, lv_ref, Hs_ref):
    Hs_ref[...] = jnp.zeros((N, H), dtype=jnp.float32)

    def step(v, _):
        Hs = Hs_ref[...]
        a_col = adjT_ref[pl.ds(v, 1), :]
        mask = (a_col == 1.0).astype(jnp.float32)
        gate = jax.nn.sigmoid(
            jax.lax.dot_general(Hs, Wg_ref[...], _DN_T) + bg_ref[...])
        mapped = jax.lax.dot_general(Hs, Wm_ref[...], _DN_T)
        agg = jnp.sum(mask.reshape(N, 1) * (gate * mapped), axis=0,
                      keepdims=True)
        agg = jnp.where(v == 0, jnp.zeros_like(agg), agg)
        x_row = types_ref[pl.ds(v, 1), :]
        gi = jax.lax.dot_general(x_row, Wih_ref[...], _DN_T) + bih_ref[...]
        gh = jax.lax.dot_general(agg, Whh_ref[...], _DN_T) + bhh_ref[...]
        r = jax.nn.sigmoid(gi[:, 0:H] + gh[:, 0:H])
        z = jax.nn.sigmoid(gi[:, H:2 * H] + gh[:, H:2 * H])
        n = jnp.tanh(gi[:, 2 * H:3 * H] + r * gh[:, 2 * H:3 * H])
        h_new = (1.0 - z) * n + z * agg
        Hs_ref[pl.ds(v, 1), :] = h_new
        return 0

    jax.lax.fori_loop(0, N, step, 0)
    hg = Hs_ref[pl.ds(N - 1, 1), :]
    mu_ref[...] = jax.lax.dot_general(hg, Wmu_ref[...], _DN_T) + bmu_ref[...]
    lv_ref[...] = jax.lax.dot_general(hg, Wlv_ref[...], _DN_T) + blv_ref[...]


@jax.jit
def kernel(g_in, W_ih, b_ih, W_hh, b_hh, Wg, bg, Wm, W_mu, b_mu, W_lv, b_lv):
    adjT = g_in[0].T
    types = g_in[1]
    mu, lv = pl.pallas_call(
        _encode_kernel,
        out_shape=[jax.ShapeDtypeStruct((1, Z), jnp.float32),
                   jax.ShapeDtypeStruct((1, Z), jnp.float32)],
        scratch_shapes=[pltpu.VMEM((N, H), jnp.float32)],
    )(adjT, types, W_ih, b_ih.reshape(1, 3 * H), W_hh, b_hh.reshape(1, 3 * H),
      Wg, bg.reshape(1, H), Wm, W_mu, b_mu.reshape(1, Z), W_lv,
      b_lv.reshape(1, Z))
    return (mu, lv)


# speedup vs baseline: 10.6800x; 1.5050x over previous
import jax
import jax.numpy as jnp
from jax.experimental import pallas as pl
from jax.experimental.pallas import tpu as pltpu

N = 200
H = 200
Z = 56

_DN_T = (((1,), (1,)), ((), ()))


def _encode_kernel(adjT_ref, types_ref, Wih_ref, bih_ref, Whh_ref, bhh_ref,
                   Wg_ref, bg_ref, Wm_ref, Wmu_ref, bmu_ref, Wlv_ref, blv_ref,
                   mu_ref, lv_ref, G_ref, GI_ref):
    GI_ref[...] = (jax.lax.dot_general(types_ref[...], Wih_ref[...], _DN_T)
                   + bih_ref[...])
    G_ref[...] = jnp.zeros((N, H), dtype=jnp.float32)

    def step(v, h_prev):
        a_col = adjT_ref[pl.ds(v, 1), :]
        mask = (a_col == 1.0).astype(jnp.float32)
        agg = jnp.sum(mask.reshape(N, 1) * G_ref[...], axis=0,
                      keepdims=True)
        gi = GI_ref[pl.ds(v, 1), :]
        gh = jax.lax.dot_general(agg, Whh_ref[...], _DN_T) + bhh_ref[...]
        r = jax.nn.sigmoid(gi[:, 0:H] + gh[:, 0:H])
        z = jax.nn.sigmoid(gi[:, H:2 * H] + gh[:, H:2 * H])
        n = jnp.tanh(gi[:, 2 * H:3 * H] + r * gh[:, 2 * H:3 * H])
        h_new = (1.0 - z) * n + z * agg
        gate = jax.nn.sigmoid(
            jax.lax.dot_general(h_new, Wg_ref[...], _DN_T) + bg_ref[...])
        mapped = jax.lax.dot_general(h_new, Wm_ref[...], _DN_T)
        G_ref[pl.ds(v, 1), :] = gate * mapped
        return h_new

    hg = jax.lax.fori_loop(0, N, step, jnp.zeros((1, H), jnp.float32))
    mu_ref[...] = jax.lax.dot_general(hg, Wmu_ref[...], _DN_T) + bmu_ref[...]
    lv_ref[...] = jax.lax.dot_general(hg, Wlv_ref[...], _DN_T) + blv_ref[...]


@jax.jit
def kernel(g_in, W_ih, b_ih, W_hh, b_hh, Wg, bg, Wm, W_mu, b_mu, W_lv, b_lv):
    adjT = g_in[0].T
    types = g_in[1]
    mu, lv = pl.pallas_call(
        _encode_kernel,
        out_shape=[jax.ShapeDtypeStruct((1, Z), jnp.float32),
                   jax.ShapeDtypeStruct((1, Z), jnp.float32)],
        scratch_shapes=[pltpu.VMEM((N, H), jnp.float32),
                        pltpu.VMEM((N, 3 * H), jnp.float32)],
    )(adjT, types, W_ih, b_ih.reshape(1, 3 * H), W_hh, b_hh.reshape(1, 3 * H),
      Wg, bg.reshape(1, H), Wm, W_mu, b_mu.reshape(1, Z), W_lv,
      b_lv.reshape(1, Z))
    return (mu, lv)
